# baseline (speedup 1.0000x reference)
import functools

import jax
import jax.numpy as jnp
from jax import lax
from jax.experimental import pallas as pl
from jax.experimental.pallas import tpu as pltpu
from jax.experimental.pallas import tpu_sc as plsc

B = 16384
F = 26
V = 100000
D = 16
DENSE = 13

NC = 2
NS = 16
NW = NC * NS
SPW = B // NW
C = 128
NCHUNK = SPW // C


def _body(x_hbm, xd_hbm, emb_hbm, lin_hbm, w_hbm, out_hbm,
          xi_v, rows_v, lin_v, xd_v, w_v, out_v, sem_e, sem_l):
    wid = lax.axis_index("s") * NC + lax.axis_index("c")
    base = wid * SPW
    pltpu.sync_copy(x_hbm.at[:, pl.ds(base, SPW)], xi_v)
    pltpu.sync_copy(xd_hbm.at[:, pl.ds(base, SPW)], xd_v)
    pltpu.sync_copy(w_hbm, w_v)

    wreg = w_v[pl.ds(0, 16)]

    def fire(f, g):
        idxsl = xi_v.at[f, pl.ds(g * C, C)]
        for d in range(D):
            pltpu.make_async_copy(
                emb_hbm.at[f * D + d].at[idxsl],
                rows_v.at[f * D + d, pl.ds(0, C)], sem_e).start()
        pltpu.make_async_copy(
            lin_hbm.at[f].at[idxsl], lin_v.at[f], sem_l).start()

    def drain(f, g):
        idxsl = xi_v.at[f, pl.ds(g * C, C)]
        for d in range(D):
            pltpu.make_async_copy(
                emb_hbm.at[f * D + d].at[idxsl],
                rows_v.at[f * D + d, pl.ds(0, C)], sem_e).wait()
        pltpu.make_async_copy(
            lin_hbm.at[f].at[idxsl], lin_v.at[f], sem_l).wait()

    DEPTH = 8

    for g in range(NCHUNK):
        def pipe_body(f, _):
            fire(f, g)

            @pl.when(f >= DEPTH)
            def _():
                drain(f - DEPTH, g)
            return 0
        lax.fori_loop(0, F, pipe_body, 0, unroll=False)

        def tail_body(f, _):
            drain(f, g)
            return 0
        lax.fori_loop(F - DEPTH, F, tail_body, 0, unroll=False)

        def grp_body(gr, _):
            sb = g * C + gr * 16
            fm = jnp.zeros((16,), jnp.float32)
            for d in range(D):
                e = rows_v[d, pl.ds(gr * 16, 16)]
                s = e
                ss = e * e
                for f in range(1, F):
                    e = rows_v[f * D + d, pl.ds(gr * 16, 16)]
                    s = s + e
                    ss = ss + e * e
                fm = fm + (s * s - ss)
            iota16 = lax.iota(jnp.int32, 16)
            zero16 = iota16 * 0
            lacc = plsc.load_gather(
                lin_v, [zero16, iota16 + gr * 16, zero16])
            for f in range(1, F):
                lacc = lacc + plsc.load_gather(
                    lin_v, [zero16 + f, iota16 + gr * 16, zero16])
            dacc = wreg[0] * xd_v[0, pl.ds(sb, 16)]
            for jj in range(1, DENSE):
                dacc = dacc + wreg[jj] * xd_v[jj, pl.ds(sb, 16)]
            out_v[pl.ds(sb, 16)] = lacc + dacc + 0.5 * fm + wreg[DENSE]
            return 0
        lax.fori_loop(0, C // 16, grp_body, 0, unroll=False)

    pltpu.sync_copy(out_v, out_hbm.at[pl.ds(base, SPW)])


@jax.jit
def _fm(x, xd, emb, lin, w):
    mesh = plsc.VectorSubcoreMesh(
        core_axis_name="c", subcore_axis_name="s",
        num_cores=NC, num_subcores=NS)
    return pl.kernel(
        _body,
        out_type=jax.ShapeDtypeStruct((B,), jnp.float32),
        mesh=mesh,
        scratch_types=[
            pltpu.VMEM((F, SPW), jnp.int32),
            pltpu.VMEM((F * D, C), jnp.float32),
            pltpu.VMEM((F, C, 1), jnp.float32),
            pltpu.VMEM((DENSE, SPW), jnp.float32),
            pltpu.VMEM((16,), jnp.float32),
            pltpu.VMEM((SPW,), jnp.float32),
            pltpu.SemaphoreType.DMA,
            pltpu.SemaphoreType.DMA,
        ],
        compiler_params=pltpu.CompilerParams(
            needs_layout_passes=False, use_tc_tiling_on_sc=False),
    )(x, xd, emb, lin, w)


def kernel(x_sparse, x_dense, emb_tables, lin_tables, lin_dense_w, bias):
    x = x_sparse.astype(jnp.int32).T
    xd = x_dense.T
    emb = jnp.swapaxes(emb_tables, 1, 2).reshape(F * D, V)
    lin = lin_tables
    w = jnp.concatenate([lin_dense_w, bias,
                         jnp.zeros((2,), jnp.float32)])
    return _fm(x, xd, emb, lin, w)


# speedup vs baseline: 6.9209x; 6.9209x over previous
import functools

import jax
import jax.numpy as jnp
from jax import lax
from jax.experimental import pallas as pl
from jax.experimental.pallas import tpu as pltpu
from jax.experimental.pallas import tpu_sc as plsc

B = 16384
F = 26
V = 100000
D = 16
DENSE = 13

NC = 2
NS = 16
NW = NC * NS
SPW = B // NW
C = 128
NCHUNK = SPW // C


def _body(x_hbm, xd_hbm, emb_hbm, lin_hbm, w_hbm, out_hbm,
          xi_v, rows_v, lin_v, xd_v, w_v, out_v, sem_e, sem_l):
    wid = lax.axis_index("s") * NC + lax.axis_index("c")
    base = wid * SPW
    pltpu.sync_copy(x_hbm.at[:, pl.ds(base, SPW)], xi_v)
    pltpu.sync_copy(xd_hbm.at[:, pl.ds(base, SPW)], xd_v)
    pltpu.sync_copy(w_hbm, w_v)

    wreg = w_v[pl.ds(0, 16)]

    def fire(f, g):
        idxsl = xi_v.at[f, pl.ds(g * C, C)]
        for d in range(D):
            pltpu.make_async_copy(
                emb_hbm.at[f * D + d].at[idxsl],
                rows_v.at[f * D + d, pl.ds(0, C)], sem_e).start()
        pltpu.make_async_copy(
            lin_hbm.at[f].at[idxsl], lin_v.at[f, pl.ds(0, C)],
            sem_l).start()

    def drain(f, g):
        idxsl = xi_v.at[f, pl.ds(g * C, C)]
        for d in range(D):
            pltpu.make_async_copy(
                emb_hbm.at[f * D + d].at[idxsl],
                rows_v.at[f * D + d, pl.ds(0, C)], sem_e).wait()
        pltpu.make_async_copy(
            lin_hbm.at[f].at[idxsl], lin_v.at[f, pl.ds(0, C)],
            sem_l).wait()

    DEPTH = 8

    for g in range(NCHUNK):
        def pipe_body(f, _):
            fire(f, g)

            @pl.when(f >= DEPTH)
            def _():
                drain(f - DEPTH, g)
            return 0
        lax.fori_loop(0, F, pipe_body, 0, unroll=False)

        def tail_body(f, _):
            drain(f, g)
            return 0
        lax.fori_loop(F - DEPTH, F, tail_body, 0, unroll=False)

        def grp_body(gr, _):
            sb = g * C + gr * 16
            fm = jnp.zeros((16,), jnp.float32)
            for d in range(D):
                e = rows_v[d, pl.ds(gr * 16, 16)]
                s = e
                ss = e * e
                for f in range(1, F):
                    e = rows_v[f * D + d, pl.ds(gr * 16, 16)]
                    s = s + e
                    ss = ss + e * e
                fm = fm + (s * s - ss)
            lacc = lin_v[0, pl.ds(gr * 16, 16)]
            for f in range(1, F):
                lacc = lacc + lin_v[f, pl.ds(gr * 16, 16)]
            dacc = wreg[0] * xd_v[0, pl.ds(sb, 16)]
            for jj in range(1, DENSE):
                dacc = dacc + wreg[jj] * xd_v[jj, pl.ds(sb, 16)]
            out_v[pl.ds(sb, 16)] = lacc + dacc + 0.5 * fm + wreg[DENSE]
            return 0
        lax.fori_loop(0, C // 16, grp_body, 0, unroll=False)

    pltpu.sync_copy(out_v, out_hbm.at[pl.ds(base, SPW)])


@jax.jit
def _fm(x, xd, emb, lin, w):
    mesh = plsc.VectorSubcoreMesh(
        core_axis_name="c", subcore_axis_name="s",
        num_cores=NC, num_subcores=NS)
    return pl.kernel(
        _body,
        out_type=jax.ShapeDtypeStruct((B,), jnp.float32),
        mesh=mesh,
        scratch_types=[
            pltpu.VMEM((F, SPW), jnp.int32),
            pltpu.VMEM((F * D, C), jnp.float32),
            pltpu.VMEM((F, C), jnp.float32),
            pltpu.VMEM((DENSE, SPW), jnp.float32),
            pltpu.VMEM((16,), jnp.float32),
            pltpu.VMEM((SPW,), jnp.float32),
            pltpu.SemaphoreType.DMA,
            pltpu.SemaphoreType.DMA,
        ],
        compiler_params=pltpu.CompilerParams(
            needs_layout_passes=False, use_tc_tiling_on_sc=False),
    )(x, xd, emb, lin, w)


def kernel(x_sparse, x_dense, emb_tables, lin_tables, lin_dense_w, bias):
    x = x_sparse.astype(jnp.int32).T
    xd = x_dense.T
    emb = jnp.swapaxes(emb_tables, 1, 2).reshape(F * D, V)
    lin = lin_tables.reshape(F, V)
    w = jnp.concatenate([lin_dense_w, bias,
                         jnp.zeros((2,), jnp.float32)])
    return _fm(x, xd, emb, lin, w)
